# baseline (device time: 105931 ns/iter reference)
import jax
import jax.numpy as jnp
from jax import lax
from jax.experimental import pallas as pl
from jax.experimental.pallas import tpu as pltpu

N_DEV = 4
F8X = jnp.float8_e4m3fn
F8W = jnp.float8_e5m2


def kernel(x, w_mat, scale_x, scale_w):
    m_global, k_per = x.shape
    _, n = w_mat.shape
    m_per = m_global // N_DEV
    n2 = n // 2

    def body(x_ref, w_ref, sx_ref, sw_ref, out_ref,
             tx_x, tx_w, rx_x, rx_w,
             tx_x_sems, tx_w_sems, rx_x_sems, rx_w_sems):
        my = lax.axis_index("i")

        barrier_sem = pltpu.get_barrier_semaphore()
        for j in range(1, N_DEV):
            peer = lax.rem(my + j, N_DEV)
            pl.semaphore_signal(barrier_sem, inc=1, device_id=(peer,),
                                device_id_type=pl.DeviceIdType.MESH)
        pl.semaphore_wait(barrier_sem, N_DEV - 1)

        sends = []

        def send(src, dst, send_sem, recv_sem, peer):
            rdma = pltpu.make_async_remote_copy(
                src_ref=src, dst_ref=dst, send_sem=send_sem,
                recv_sem=recv_sem, device_id=(peer,),
                device_id_type=pl.DeviceIdType.MESH,
            )
            rdma.start()
            sends.append(rdma)

        for h in (0, 1):
            cols = pl.ds(h * n2, n2)
            tx_w[:, cols] = w_ref[:, cols].astype(F8W)
            for j in range(1, N_DEV):
                peer = lax.rem(my + j, N_DEV)
                send(tx_w.at[:, cols],
                     rx_w.at[N_DEV - 1 - j, :, cols],
                     tx_w_sems.at[j - 1, h], rx_w_sems.at[N_DEV - 1 - j, h],
                     peer)
        for j in range(1, N_DEV):
            peer = lax.rem(my + j, N_DEV)
            tx_x[j - 1, :, :] = x_ref[pl.ds(peer * m_per, m_per), :].astype(F8X)
            send(tx_x.at[j - 1], rx_x.at[N_DEV - 1 - j],
                 tx_x_sems.at[j - 1], rx_x_sems.at[N_DEV - 1 - j], peer)

        out_ref[:, :] = jnp.dot(
            x_ref[pl.ds(my * m_per, m_per), :].astype(jnp.bfloat16),
            w_ref[:, :].astype(jnp.bfloat16),
            preferred_element_type=jnp.float32,
        )

        def wait_recv(dst, sem):
            rdma = pltpu.make_async_remote_copy(
                src_ref=dst, dst_ref=dst, send_sem=tx_x_sems.at[0],
                recv_sem=sem, device_id=(my,),
                device_id_type=pl.DeviceIdType.MESH,
            )
            rdma.wait_recv()

        xb = {}
        for slot in (0, 2, 1):
            wait_recv(rx_x.at[slot], rx_x_sems.at[slot])
            xb[slot] = rx_x[slot, :, :].astype(jnp.bfloat16)

        for slot in (0, 2, 1):
            for h in (0, 1):
                wait_recv(rx_w.at[slot, :, pl.ds(h * n2, n2)],
                          rx_w_sems.at[slot, h])
                contrib = jnp.dot(
                    xb[slot],
                    rx_w[slot, :, pl.ds(h * n2, n2)].astype(jnp.bfloat16),
                    preferred_element_type=jnp.float32,
                )
                cols = pl.ds(h * n2, n2)
                if slot == 1:
                    out_ref[:, cols] = (
                        (out_ref[:, cols] + contrib) * (sx_ref[0] * sw_ref[0])
                    )
                else:
                    out_ref[:, cols] = out_ref[:, cols] + contrib

        for s in sends:
            s.wait_send()

    return pl.pallas_call(
        body,
        out_shape=jax.ShapeDtypeStruct((m_per, n), jnp.float32),
        in_specs=[
            pl.BlockSpec(memory_space=pltpu.VMEM),
            pl.BlockSpec(memory_space=pltpu.VMEM),
            pl.BlockSpec(memory_space=pltpu.SMEM),
            pl.BlockSpec(memory_space=pltpu.SMEM),
        ],
        out_specs=pl.BlockSpec(memory_space=pltpu.VMEM),
        scratch_shapes=[
            pltpu.VMEM((N_DEV - 1, m_per, k_per), F8X),
            pltpu.VMEM((k_per, n), F8W),
            pltpu.VMEM((N_DEV - 1, m_per, k_per), F8X),
            pltpu.VMEM((N_DEV - 1, k_per, n), F8W),
            pltpu.SemaphoreType.DMA((N_DEV - 1,)),
            pltpu.SemaphoreType.DMA((N_DEV - 1, 2)),
            pltpu.SemaphoreType.DMA((N_DEV - 1,)),
            pltpu.SemaphoreType.DMA((N_DEV - 1, 2)),
        ],
        compiler_params=pltpu.CompilerParams(
            collective_id=0,
            vmem_limit_bytes=62 * 1024 * 1024,
        ),
    )(x, w_mat, scale_x, scale_w)


# device time: 94214 ns/iter; 1.1244x vs baseline; 1.1244x over previous
import jax
import jax.numpy as jnp
from jax import lax
from jax.experimental import pallas as pl
from jax.experimental.pallas import tpu as pltpu

N_DEV = 4
F8X = jnp.float8_e4m3fn
F8W = jnp.float8_e5m2


def kernel(x, w_mat, scale_x, scale_w):
    m_global, k_per = x.shape
    _, n = w_mat.shape
    m_per = m_global // N_DEV
    n2 = n // 2

    def body(x_ref, w_ref, sx_ref, sw_ref, out_ref,
             tx_x, tx_w, rx_x, rx_w,
             tx_x_sems, tx_w_sems, rx_x_sems, rx_w_sems):
        my = lax.axis_index("i")

        barrier_sem = pltpu.get_barrier_semaphore()
        for j in range(1, N_DEV):
            peer = lax.rem(my + j, N_DEV)
            pl.semaphore_signal(barrier_sem, inc=1, device_id=(peer,),
                                device_id_type=pl.DeviceIdType.MESH)
        pl.semaphore_wait(barrier_sem, N_DEV - 1)

        def send(src, dst, send_sem, recv_sem, peer, into):
            rdma = pltpu.make_async_remote_copy(
                src_ref=src, dst_ref=dst, send_sem=send_sem,
                recv_sem=recv_sem, device_id=(peer,),
                device_id_type=pl.DeviceIdType.MESH,
            )
            rdma.start()
            into.append(rdma)

        def send_payload(j, into):
            peer = lax.rem(my + j, N_DEV)
            send(tx_x.at[j - 1], rx_x.at[N_DEV - 1 - j],
                 tx_x_sems.at[j - 1], rx_x_sems.at[N_DEV - 1 - j], peer, into)
            for h in (0, 1):
                cols = pl.ds(h * n2, n2)
                send(tx_w.at[:, cols], rx_w.at[N_DEV - 1 - j, :, cols],
                     tx_w_sems.at[j - 1, h], rx_w_sems.at[N_DEV - 1 - j, h],
                     peer, into)

        direct, diag = [], []
        for h in (0, 1):
            cols = pl.ds(h * n2, n2)
            tx_w[:, cols] = w_ref[:, cols].astype(F8W)
        for j in (1, 3):
            peer = lax.rem(my + j, N_DEV)
            tx_x[j - 1, :, :] = x_ref[pl.ds(peer * m_per, m_per), :].astype(F8X)
            send_payload(j, direct)
        diag_peer = lax.rem(my + 2, N_DEV)
        tx_x[1, :, :] = x_ref[pl.ds(diag_peer * m_per, m_per), :].astype(F8X)

        out_ref[:, :] = jnp.dot(
            x_ref[pl.ds(my * m_per, m_per), :].astype(jnp.bfloat16),
            w_ref[:, :].astype(jnp.bfloat16),
            preferred_element_type=jnp.float32,
        )

        for s in direct:
            s.wait_send()
        send_payload(2, diag)

        def wait_recv(dst, sem):
            rdma = pltpu.make_async_remote_copy(
                src_ref=dst, dst_ref=dst, send_sem=tx_x_sems.at[0],
                recv_sem=sem, device_id=(my,),
                device_id_type=pl.DeviceIdType.MESH,
            )
            rdma.wait_recv()

        for slot in (0, 2, 1):
            wait_recv(rx_x.at[slot], rx_x_sems.at[slot])
            xb = rx_x[slot, :, :].astype(jnp.bfloat16)
            for h in (0, 1):
                cols = pl.ds(h * n2, n2)
                wait_recv(rx_w.at[slot, :, cols], rx_w_sems.at[slot, h])
                contrib = jnp.dot(
                    xb, rx_w[slot, :, cols].astype(jnp.bfloat16),
                    preferred_element_type=jnp.float32,
                )
                if slot == 1:
                    out_ref[:, cols] = (
                        (out_ref[:, cols] + contrib) * (sx_ref[0] * sw_ref[0])
                    )
                else:
                    out_ref[:, cols] = out_ref[:, cols] + contrib

        for s in diag:
            s.wait_send()

    return pl.pallas_call(
        body,
        out_shape=jax.ShapeDtypeStruct((m_per, n), jnp.float32),
        in_specs=[
            pl.BlockSpec(memory_space=pltpu.VMEM),
            pl.BlockSpec(memory_space=pltpu.VMEM),
            pl.BlockSpec(memory_space=pltpu.SMEM),
            pl.BlockSpec(memory_space=pltpu.SMEM),
        ],
        out_specs=pl.BlockSpec(memory_space=pltpu.VMEM),
        scratch_shapes=[
            pltpu.VMEM((N_DEV - 1, m_per, k_per), F8X),
            pltpu.VMEM((k_per, n), F8W),
            pltpu.VMEM((N_DEV - 1, m_per, k_per), F8X),
            pltpu.VMEM((N_DEV - 1, k_per, n), F8W),
            pltpu.SemaphoreType.DMA((N_DEV - 1,)),
            pltpu.SemaphoreType.DMA((N_DEV - 1, 2)),
            pltpu.SemaphoreType.DMA((N_DEV - 1,)),
            pltpu.SemaphoreType.DMA((N_DEV - 1, 2)),
        ],
        compiler_params=pltpu.CompilerParams(
            collective_id=0,
            vmem_limit_bytes=62 * 1024 * 1024,
        ),
    )(x, w_mat, scale_x, scale_w)


# device time: 93881 ns/iter; 1.1284x vs baseline; 1.0035x over previous
import jax
import jax.numpy as jnp
from jax import lax
from jax.experimental import pallas as pl
from jax.experimental.pallas import tpu as pltpu

N_DEV = 4
F8X = jnp.float8_e4m3fn
F8W = jnp.float8_e5m2


def kernel(x, w_mat, scale_x, scale_w):
    m_global, k_per = x.shape
    _, n = w_mat.shape
    m_per = m_global // N_DEV
    n2 = n // 2

    def body(x_ref, w_ref, sx_ref, sw_ref, out_ref,
             tx_x, tx_w, rx_x, rx_w,
             tx_x_sems, tx_w_sems, rx_x_sems, rx_w_sems):
        my = lax.axis_index("i")

        barrier_sem = pltpu.get_barrier_semaphore()
        for j in range(1, N_DEV):
            peer = lax.rem(my + j, N_DEV)
            pl.semaphore_signal(barrier_sem, inc=1, device_id=(peer,),
                                device_id_type=pl.DeviceIdType.MESH)
        pl.semaphore_wait(barrier_sem, N_DEV - 1)

        def send(src, dst, send_sem, recv_sem, peer, into):
            rdma = pltpu.make_async_remote_copy(
                src_ref=src, dst_ref=dst, send_sem=send_sem,
                recv_sem=recv_sem, device_id=(peer,),
                device_id_type=pl.DeviceIdType.MESH,
            )
            rdma.start()
            into.append(rdma)

        def send_x(j, into):
            peer = lax.rem(my + j, N_DEV)
            send(tx_x.at[j - 1], rx_x.at[N_DEV - 1 - j],
                 tx_x_sems.at[j - 1], rx_x_sems.at[N_DEV - 1 - j], peer, into)

        def send_w_half(j, h, into):
            peer = lax.rem(my + j, N_DEV)
            cols = pl.ds(h * n2, n2)
            send(tx_w.at[:, cols], rx_w.at[N_DEV - 1 - j, :, cols],
                 tx_w_sems.at[j - 1, h], rx_w_sems.at[N_DEV - 1 - j, h],
                 peer, into)

        def send_payload(j, into):
            send_x(j, into)
            for h in (0, 1):
                send_w_half(j, h, into)

        direct, diag = [], []
        for j in (1, 3):
            peer = lax.rem(my + j, N_DEV)
            tx_x[j - 1, :, :] = x_ref[pl.ds(peer * m_per, m_per), :].astype(F8X)
            send_x(j, direct)
        for h in (0, 1):
            cols = pl.ds(h * n2, n2)
            tx_w[:, cols] = w_ref[:, cols].astype(F8W)
            for j in (1, 3):
                send_w_half(j, h, direct)
        diag_peer = lax.rem(my + 2, N_DEV)
        tx_x[1, :, :] = x_ref[pl.ds(diag_peer * m_per, m_per), :].astype(F8X)

        out_ref[:, :] = jnp.dot(
            x_ref[pl.ds(my * m_per, m_per), :].astype(jnp.bfloat16),
            w_ref[:, :].astype(jnp.bfloat16),
            preferred_element_type=jnp.float32,
        )

        for s in direct:
            s.wait_send()
        send_payload(2, diag)

        def wait_recv(dst, sem):
            rdma = pltpu.make_async_remote_copy(
                src_ref=dst, dst_ref=dst, send_sem=tx_x_sems.at[0],
                recv_sem=sem, device_id=(my,),
                device_id_type=pl.DeviceIdType.MESH,
            )
            rdma.wait_recv()

        for slot in (0, 2, 1):
            wait_recv(rx_x.at[slot], rx_x_sems.at[slot])
            xb = rx_x[slot, :, :].astype(jnp.bfloat16)
            for h in (0, 1):
                cols = pl.ds(h * n2, n2)
                wait_recv(rx_w.at[slot, :, cols], rx_w_sems.at[slot, h])
                contrib = jnp.dot(
                    xb, rx_w[slot, :, cols].astype(jnp.bfloat16),
                    preferred_element_type=jnp.float32,
                )
                if slot == 1:
                    out_ref[:, cols] = (
                        (out_ref[:, cols] + contrib) * (sx_ref[0] * sw_ref[0])
                    )
                else:
                    out_ref[:, cols] = out_ref[:, cols] + contrib

        for s in diag:
            s.wait_send()

    return pl.pallas_call(
        body,
        out_shape=jax.ShapeDtypeStruct((m_per, n), jnp.float32),
        in_specs=[
            pl.BlockSpec(memory_space=pltpu.VMEM),
            pl.BlockSpec(memory_space=pltpu.VMEM),
            pl.BlockSpec(memory_space=pltpu.SMEM),
            pl.BlockSpec(memory_space=pltpu.SMEM),
        ],
        out_specs=pl.BlockSpec(memory_space=pltpu.VMEM),
        scratch_shapes=[
            pltpu.VMEM((N_DEV - 1, m_per, k_per), F8X),
            pltpu.VMEM((k_per, n), F8W),
            pltpu.VMEM((N_DEV - 1, m_per, k_per), F8X),
            pltpu.VMEM((N_DEV - 1, k_per, n), F8W),
            pltpu.SemaphoreType.DMA((N_DEV - 1,)),
            pltpu.SemaphoreType.DMA((N_DEV - 1, 2)),
            pltpu.SemaphoreType.DMA((N_DEV - 1,)),
            pltpu.SemaphoreType.DMA((N_DEV - 1, 2)),
        ],
        compiler_params=pltpu.CompilerParams(
            collective_id=0,
            vmem_limit_bytes=62 * 1024 * 1024,
        ),
    )(x, w_mat, scale_x, scale_w)


# device time: 93804 ns/iter; 1.1293x vs baseline; 1.0008x over previous
import jax
import jax.numpy as jnp
from jax import lax
from jax.experimental import pallas as pl
from jax.experimental.pallas import tpu as pltpu

N_DEV = 4
F8X = jnp.float8_e4m3fn
F8W = jnp.float8_e5m2


def kernel(x, w_mat, scale_x, scale_w):
    m_global, k_per = x.shape
    _, n = w_mat.shape
    m_per = m_global // N_DEV
    n2 = n // 2

    def body(x_ref, w_ref, sx_ref, sw_ref, out_ref,
             tx_x, tx_w, rx_x, rx_w,
             tx_x_sems, tx_w_sems, rx_x_sems, rx_w_sems):
        my = lax.axis_index("i")

        barrier_sem = pltpu.get_barrier_semaphore()
        for j in range(1, N_DEV):
            peer = lax.rem(my + j, N_DEV)
            pl.semaphore_signal(barrier_sem, inc=1, device_id=(peer,),
                                device_id_type=pl.DeviceIdType.MESH)
        pl.semaphore_wait(barrier_sem, N_DEV - 1)

        def send(src, dst, send_sem, recv_sem, peer, into):
            rdma = pltpu.make_async_remote_copy(
                src_ref=src, dst_ref=dst, send_sem=send_sem,
                recv_sem=recv_sem, device_id=(peer,),
                device_id_type=pl.DeviceIdType.MESH,
            )
            rdma.start()
            into.append(rdma)

        def send_x(j, into):
            peer = lax.rem(my + j, N_DEV)
            send(tx_x.at[j - 1], rx_x.at[N_DEV - 1 - j],
                 tx_x_sems.at[j - 1], rx_x_sems.at[N_DEV - 1 - j], peer, into)

        def send_w_half(j, h, into):
            peer = lax.rem(my + j, N_DEV)
            cols = pl.ds(h * n2, n2)
            send(tx_w.at[:, cols], rx_w.at[N_DEV - 1 - j, :, cols],
                 tx_w_sems.at[j - 1, h], rx_w_sems.at[N_DEV - 1 - j, h],
                 peer, into)

        def send_payload(j, into):
            send_x(j, into)
            for h in (0, 1):
                send_w_half(j, h, into)

        direct, diag = [], []
        for j in (1, 3):
            peer = lax.rem(my + j, N_DEV)
            tx_x[j - 1, :, :] = x_ref[pl.ds(peer * m_per, m_per), :].astype(F8X)
            send_x(j, direct)
        for h in (0, 1):
            cols = pl.ds(h * n2, n2)
            tx_w[:, cols] = w_ref[:, cols].astype(F8W)
            for j in (1, 3):
                send_w_half(j, h, direct)
        diag_peer = lax.rem(my + 2, N_DEV)
        tx_x[1, :, :] = x_ref[pl.ds(diag_peer * m_per, m_per), :].astype(F8X)

        out_ref[:, :] = jnp.dot(
            x_ref[pl.ds(my * m_per, m_per), :].astype(jnp.bfloat16),
            w_ref[:, :].astype(jnp.bfloat16),
            preferred_element_type=jnp.float32,
        )

        for s in direct:
            s.wait_send()
        send_payload(2, diag)

        def wait_recv(dst, sem):
            rdma = pltpu.make_async_remote_copy(
                src_ref=dst, dst_ref=dst, send_sem=tx_x_sems.at[0],
                recv_sem=sem, device_id=(my,),
                device_id_type=pl.DeviceIdType.MESH,
            )
            rdma.wait_recv()

        for slot in (0, 2):
            wait_recv(rx_x.at[slot], rx_x_sems.at[slot])
        xb0 = rx_x[0, :, :].astype(jnp.bfloat16)
        xb2 = rx_x[2, :, :].astype(jnp.bfloat16)
        for h in (0, 1):
            cols = pl.ds(h * n2, n2)
            wait_recv(rx_w.at[0, :, cols], rx_w_sems.at[0, h])
            wait_recv(rx_w.at[2, :, cols], rx_w_sems.at[2, h])
            contrib = jnp.dot(
                xb0, rx_w[0, :, cols].astype(jnp.bfloat16),
                preferred_element_type=jnp.float32,
            ) + jnp.dot(
                xb2, rx_w[2, :, cols].astype(jnp.bfloat16),
                preferred_element_type=jnp.float32,
            )
            out_ref[:, cols] = out_ref[:, cols] + contrib

        wait_recv(rx_x.at[1], rx_x_sems.at[1])
        xb1 = rx_x[1, :, :].astype(jnp.bfloat16)
        for h in (0, 1):
            cols = pl.ds(h * n2, n2)
            wait_recv(rx_w.at[1, :, cols], rx_w_sems.at[1, h])
            contrib = jnp.dot(
                xb1, rx_w[1, :, cols].astype(jnp.bfloat16),
                preferred_element_type=jnp.float32,
            )
            out_ref[:, cols] = (
                (out_ref[:, cols] + contrib) * (sx_ref[0] * sw_ref[0])
            )

        for s in diag:
            s.wait_send()

    return pl.pallas_call(
        body,
        out_shape=jax.ShapeDtypeStruct((m_per, n), jnp.float32),
        in_specs=[
            pl.BlockSpec(memory_space=pltpu.VMEM),
            pl.BlockSpec(memory_space=pltpu.VMEM),
            pl.BlockSpec(memory_space=pltpu.SMEM),
            pl.BlockSpec(memory_space=pltpu.SMEM),
        ],
        out_specs=pl.BlockSpec(memory_space=pltpu.VMEM),
        scratch_shapes=[
            pltpu.VMEM((N_DEV - 1, m_per, k_per), F8X),
            pltpu.VMEM((k_per, n), F8W),
            pltpu.VMEM((N_DEV - 1, m_per, k_per), F8X),
            pltpu.VMEM((N_DEV - 1, k_per, n), F8W),
            pltpu.SemaphoreType.DMA((N_DEV - 1,)),
            pltpu.SemaphoreType.DMA((N_DEV - 1, 2)),
            pltpu.SemaphoreType.DMA((N_DEV - 1,)),
            pltpu.SemaphoreType.DMA((N_DEV - 1, 2)),
        ],
        compiler_params=pltpu.CompilerParams(
            collective_id=0,
            vmem_limit_bytes=62 * 1024 * 1024,
        ),
    )(x, w_mat, scale_x, scale_w)
